# Initial kernel scaffold; baseline (speedup 1.0000x reference)
import jax
import jax.numpy as jnp
from jax.experimental import pallas as pl


def kernel(points, W, b, gamma, beta):
    raise NotImplementedError("write your pallas kernel here")


# speedup vs baseline: 25.7790x; 25.7790x over previous
import functools

import jax
import jax.numpy as jnp
from jax import lax
from jax.experimental import pallas as pl
from jax.experimental.pallas import tpu as pltpu
from jax.experimental.pallas import tpu_sc as plsc

_B, _N, _C, _K, _OUT = 4, 4096, 64, 20, 128
_RT = 256
_NRT = _N // _RT
_NBLK = _N // 128
_S = 5
_KPAD = 128
_IMAX = 0x7FFFFFFF


def _tc_body(pts_ref, ptsT_ref, wa_ref, wb_ref, c0_ref, idx_ref, a_ref, bv_ref):
    bidx = pl.program_id(0)
    pr = pts_ref[0]
    sqr = jnp.sum(pr * pr, axis=1, keepdims=True)

    a_ref[0] = jnp.dot(pr, wa_ref[...], preferred_element_type=jnp.float32)
    bv_ref[0] = (jnp.dot(pr, wb_ref[...], preferred_element_type=jnp.float32)
                 + c0_ref[0:1, :])

    M = [jnp.full((_RT, 128), 3.0e38, jnp.float32) for _ in range(_S)]
    for blk in range(_NBLK):
        pb = ptsT_ref[0, :, blk * 128:(blk + 1) * 128]
        dot = lax.dot_general(pr, pb, (((1,), (0,)), ((), ())),
                              preferred_element_type=jnp.float32)
        sqc1 = jnp.sum(pb * pb, axis=0, keepdims=True) + 1.0
        d1 = jnp.maximum(sqr - 2.0 * dot + sqc1, 1.0)
        bits = lax.bitcast_convert_type(d1, jnp.int32)
        key = (bits & jnp.int32(~31)) | jnp.int32(blk)
        t = lax.bitcast_convert_type(key, jnp.float32)
        for j in range(_S):
            lo = jnp.minimum(M[j], t)
            t = jnp.maximum(M[j], t)
            M[j] = lo

    lane_f = lax.broadcasted_iota(jnp.int32, (_RT, 128), 1).astype(jnp.float32)
    cols = []
    for _ in range(_K):
        v = jnp.min(M[0], axis=1, keepdims=True)
        eq = M[0] == v
        lf = jnp.min(jnp.where(eq, lane_f, jnp.float32(128.0)), axis=1,
                     keepdims=True)
        vb = lax.bitcast_convert_type(v, jnp.int32)
        cols.append((vb & jnp.int32(31)) * 128 + lf.astype(jnp.int32))
        onehot = lane_f == lf
        for j in range(_S - 1):
            M[j] = jnp.where(onehot, M[j + 1], M[j])
        M[_S - 1] = jnp.where(onehot, jnp.float32(3.0e38), M[_S - 1])

    idx_blk = jnp.concatenate(
        cols + [jnp.zeros((_RT, _KPAD - _K), jnp.int32)], axis=1)
    idx_ref[0, 0] = idx_blk + bidx * _N


_TC_GRID = (_B, _NRT)
_TC_IN_SPECS = [
    pl.BlockSpec((1, _RT, _C), lambda b, r: (b, r, 0)),
    pl.BlockSpec((1, _C, _N), lambda b, r: (b, 0, 0)),
    pl.BlockSpec((_C, _OUT), lambda b, r: (0, 0)),
    pl.BlockSpec((_C, _OUT), lambda b, r: (0, 0)),
    pl.BlockSpec((8, _OUT), lambda b, r: (0, 0)),
]
_TC_OUT_SPECS = [
    pl.BlockSpec((1, 1, _RT, _KPAD), lambda b, r: (b, r, 0, 0)),
    pl.BlockSpec((1, _RT, _OUT), lambda b, r: (b, r, 0)),
    pl.BlockSpec((1, _RT, _OUT), lambda b, r: (b, r, 0)),
]
_TC_OUT_SHAPE = [
    jax.ShapeDtypeStruct((_B, _NRT, _RT, _KPAD), jnp.int32),
    jax.ShapeDtypeStruct((_B, _N, _OUT), jnp.float32),
    jax.ShapeDtypeStruct((_B, _N, _OUT), jnp.float32),
]


_NW = 32
_PW = _B * _N // _NW
_CH = 8
_NCH = _PW // _CH
_HK = _CH * _K // 2


def _sc_body(a_hbm, idx_hbm, bv_hbm, out_hbm, idx_v, rows_v, bv_v, out_v, sem):
    w = lax.axis_index("s") * 2 + lax.axis_index("c")

    def chunk(ch, carry):
        base = w * _PW + ch * _CH
        ib = base * _K
        pltpu.sync_copy(idx_hbm.at[pl.ds(ib, _HK)], idx_v.at[0])
        pltpu.sync_copy(idx_hbm.at[pl.ds(ib + _HK, _HK)], idx_v.at[1])
        cp1 = pltpu.async_copy(a_hbm.at[idx_v.at[0]],
                               rows_v.at[pl.ds(0, _HK)], sem)
        cp2 = pltpu.async_copy(a_hbm.at[idx_v.at[1]],
                               rows_v.at[pl.ds(_HK, _HK)], sem)
        pltpu.sync_copy(bv_hbm.at[pl.ds(base, _CH)], bv_v)
        cp1.wait()
        cp2.wait()
        for i in range(_CH):
            for c in range(8):
                sl = pl.ds(c * 16, 16)
                acc = rows_v[i * _K, sl]
                for k in range(1, _K):
                    acc = jnp.maximum(acc, rows_v[i * _K + k, sl])
                acc = acc + bv_v[i, sl]
                out_v[i, sl] = jnp.maximum(acc, 0.0)
        pltpu.sync_copy(out_v, out_hbm.at[pl.ds(base, _CH)])
        return carry

    lax.fori_loop(0, _NCH, chunk, 0)


@functools.cache
def _sc_gather_max():
    return pl.kernel(
        _sc_body,
        mesh=plsc.VectorSubcoreMesh(core_axis_name="c", subcore_axis_name="s"),
        out_type=jax.ShapeDtypeStruct((_B * _N, _OUT), jnp.float32),
        scratch_types=[
            pltpu.VMEM((2, _HK), jnp.int32),
            pltpu.VMEM((_CH * _K, _OUT), jnp.float32),
            pltpu.VMEM((_CH, _OUT), jnp.float32),
            pltpu.VMEM((_CH, _OUT), jnp.float32),
            pltpu.SemaphoreType.DMA,
        ],
    )


def kernel(points, W, b, gamma, beta):
    W1 = W[:, :_C]
    W2 = W[:, _C:]
    wa = (W1 * gamma[:, None]).T
    wb = ((W2 - W1) * gamma[:, None]).T
    c0 = jnp.broadcast_to((b * gamma + beta)[None, :], (8, _OUT))
    ptsT = jnp.swapaxes(points, 1, 2)

    idx4, a, bv = pl.pallas_call(
        _tc_body,
        grid=_TC_GRID,
        in_specs=_TC_IN_SPECS,
        out_specs=_TC_OUT_SPECS,
        out_shape=_TC_OUT_SHAPE,
    )(points, ptsT, wa, wb, c0)

    idx_flat = idx4.reshape(_B, _N, _KPAD)[:, :, :_K].reshape(-1)
    out = _sc_gather_max()(a.reshape(_B * _N, _OUT), idx_flat,
                           bv.reshape(_B * _N, _OUT))
    return out.reshape(_B, _N, _OUT)
